# baseline (speedup 1.0000x reference)
import jax
import jax.numpy as jnp
from jax import lax
from jax.experimental import pallas as pl
from jax.experimental.pallas import tpu as pltpu
from jax.experimental.pallas import tpu_sc as plsc

N_POINTS = 262144
IN_DIM = 3
N_LEVELS = 16
F_PER_LEVEL = 2
LOG2_T = 19
T = 1 << LOG2_T
BASE_RES = 16
PER_LEVEL_SCALE = 1.3819128799
P1 = 2654435761
P2 = 805459861
OUT_DIM = IN_DIM + N_LEVELS * F_PER_LEVEL


def _res_floor(l):
    import numpy as np
    return int(np.floor(BASE_RES * (PER_LEVEL_SCALE ** l)))


RES = [_res_floor(l) for l in range(N_LEVELS)]
DENSE = [(r + 1) ** 3 <= T for r in RES]

NW = 32
PTS_PER_W = N_POINTS // NW
C = 1024
NCHUNK = PTS_PER_W // C
NV = C // 16


def _body(x0, x1, x2, table, out, xbuf, fracbuf, idxbuf, featbuf, outbuf, sem0, sem1):
    xs = (x0, x1, x2)
    cid = lax.axis_index("c")
    sid = lax.axis_index("s")
    wid = sid * 2 + cid
    iota = lax.iota(jnp.int32, 16)
    iota35 = iota * OUT_DIM
    iota2 = iota * 2

    def chunk_body(ch, carry):
        base = wid * PTS_PER_W + ch * C
        for d in range(IN_DIM):
            pltpu.sync_copy(xs[d].at[pl.ds(base, C)], xbuf.at[pl.ds(d * C, C)])

        def xcol_body(i, c2):
            fo = i * (16 * OUT_DIM)
            for d in range(IN_DIM):
                v = xbuf[pl.ds(d * C + i * 16, 16)]
                plsc.store_scatter(outbuf, [iota35 + (fo + d)], v)
            return c2
        lax.fori_loop(0, NV, xcol_body, 0)

        def passA(l):
            p = l & 1
            resf = jnp.float32(RES[l])

            def body_i(i, c2):
                o = i * 16
                ip = []
                for d in range(IN_DIM):
                    pos = xbuf[pl.ds(d * C + o, 16)] * resf
                    ipd = pos.astype(jnp.int32)
                    fracbuf[pl.ds((p * IN_DIM + d) * C + o, 16)] = (
                        pos - ipd.astype(jnp.float32))
                    ip.append(ipd)
                if DENSE[l]:
                    s = RES[l] + 1
                    h0 = (ip[0], ip[0] + 1)
                    h1 = (ip[1] * s, ip[1] * s + s)
                    b2 = ip[2] * (s * s) + (l * T)
                    h2 = (b2, b2 + s * s)
                    t01 = [h0[0] + h1[0], h0[1] + h1[0], h0[0] + h1[1], h0[1] + h1[1]]
                    for c in range(8):
                        e = t01[c & 3] + h2[(c >> 2) & 1]
                        e = e + e
                        idxbuf[pl.ds((p * 16 + 2 * c) * C + o, 16)] = e
                        idxbuf[pl.ds((p * 16 + 2 * c + 1) * C + o, 16)] = e + 1
                else:
                    u = [plsc.bitcast(v, jnp.uint32) for v in ip]
                    h0 = (u[0], u[0] + jnp.uint32(1))
                    m1 = u[1] * jnp.uint32(P1)
                    h1 = (m1, m1 + jnp.uint32(P1))
                    m2 = u[2] * jnp.uint32(P2)
                    h2 = (m2, m2 + jnp.uint32(P2))
                    t01 = [h0[0] ^ h1[0], h0[1] ^ h1[0], h0[0] ^ h1[1], h0[1] ^ h1[1]]
                    mask = jnp.uint32(T - 1)
                    for c in range(8):
                        hv = t01[c & 3] ^ h2[(c >> 2) & 1]
                        e = plsc.bitcast(hv & mask, jnp.int32) + (l * T)
                        e = e + e
                        idxbuf[pl.ds((p * 16 + 2 * c) * C + o, 16)] = e
                        idxbuf[pl.ds((p * 16 + 2 * c + 1) * C + o, 16)] = e + 1
                return c2
            lax.fori_loop(0, NV, body_i, 0)

        def fire(l):
            p = l & 1
            sem = sem0 if p == 0 else sem1
            handles = []
            for k in range(16):
                q = (p * 16 + k) * C
                handles.append(
                    pltpu.async_copy(
                        table.at[idxbuf.at[pl.ds(q, C)]],
                        featbuf.at[pl.ds(q, C)],
                        sem))
            return handles

        def passB(l):
            p = l & 1
            col = IN_DIM + 2 * l

            def body_i(i, c2):
                o = i * 16
                fo = i * (16 * OUT_DIM)
                f0 = fracbuf[pl.ds((p * IN_DIM + 0) * C + o, 16)]
                f1 = fracbuf[pl.ds((p * IN_DIM + 1) * C + o, 16)]
                f2 = fracbuf[pl.ds((p * IN_DIM + 2) * C + o, 16)]
                g0 = (1.0 - f0, f0)
                g1 = (1.0 - f1, f1)
                g2 = (1.0 - f2, f2)
                w01 = [g0[0] * g1[0], g0[1] * g1[0], g0[0] * g1[1], g0[1] * g1[1]]
                acc0 = jnp.zeros((16,), jnp.float32)
                acc1 = jnp.zeros((16,), jnp.float32)
                for c in range(8):
                    w = w01[c & 3] * g2[(c >> 2) & 1]
                    v0 = featbuf[pl.ds((p * 16 + 2 * c) * C + o, 16)]
                    v1 = featbuf[pl.ds((p * 16 + 2 * c + 1) * C + o, 16)]
                    acc0 = acc0 + w * v0
                    acc1 = acc1 + w * v1
                plsc.store_scatter(outbuf, [iota35 + (fo + col)], acc0)
                plsc.store_scatter(outbuf, [iota35 + (fo + col + 1)], acc1)
                return c2
            lax.fori_loop(0, NV, body_i, 0)

        passA(0)
        pending = fire(0)
        for l in range(1, N_LEVELS + 1):
            if l < N_LEVELS:
                passA(l)
                nxt = fire(l)
            for h in pending:
                h.wait()
            passB(l - 1)
            if l < N_LEVELS:
                pending = nxt

        pltpu.sync_copy(outbuf, out.at[pl.ds(base * OUT_DIM, C * OUT_DIM)])
        return carry

    lax.fori_loop(0, NCHUNK, chunk_body, 0)


@jax.jit
def kernel(x, grid):
    x0, x1, x2 = x[:, 0], x[:, 1], x[:, 2]
    table = grid.reshape(N_LEVELS * T * F_PER_LEVEL)
    mesh = plsc.VectorSubcoreMesh(core_axis_name="c", subcore_axis_name="s")
    f = pl.kernel(
        _body,
        out_type=jax.ShapeDtypeStruct((N_POINTS * OUT_DIM,), jnp.float32),
        mesh=mesh,
        compiler_params=pltpu.CompilerParams(needs_layout_passes=False),
        scratch_types=[
            pltpu.VMEM((IN_DIM * C,), jnp.float32),
            pltpu.VMEM((2 * IN_DIM * C,), jnp.float32),
            pltpu.VMEM((2 * 16 * C,), jnp.int32),
            pltpu.VMEM((2 * 16 * C,), jnp.float32),
            pltpu.VMEM((C * OUT_DIM,), jnp.float32),
            pltpu.SemaphoreType.DMA,
            pltpu.SemaphoreType.DMA,
        ],
    )
    return f(x0, x1, x2, table).reshape(N_POINTS, OUT_DIM)


# speedup vs baseline: 1.0068x; 1.0068x over previous
import jax
import jax.numpy as jnp
from jax import lax
from jax.experimental import pallas as pl
from jax.experimental.pallas import tpu as pltpu
from jax.experimental.pallas import tpu_sc as plsc

N_POINTS = 262144
IN_DIM = 3
N_LEVELS = 16
F_PER_LEVEL = 2
LOG2_T = 19
T = 1 << LOG2_T
BASE_RES = 16
PER_LEVEL_SCALE = 1.3819128799
P1 = 2654435761
P2 = 805459861
ENC_DIM = N_LEVELS * F_PER_LEVEL
OUT_DIM = IN_DIM + ENC_DIM


def _res_floor(l):
    import numpy as np
    return int(np.floor(BASE_RES * (PER_LEVEL_SCALE ** l)))


RES = [_res_floor(l) for l in range(N_LEVELS)]
DENSE = [(r + 1) ** 3 <= T for r in RES]

NW = 32
PTS_PER_W = N_POINTS // NW
C = 1024
NCHUNK = PTS_PER_W // C
NV = C // 16


def _body(x0, x1, x2, table, out, xbuf, fracbuf, idxbuf, featbuf, outbuf, sem0, sem1):
    xs = (x0, x1, x2)
    cid = lax.axis_index("c")
    sid = lax.axis_index("s")
    wid = sid * 2 + cid

    def chunk_body(ch, carry):
        base = wid * PTS_PER_W + ch * C
        for d in range(IN_DIM):
            pltpu.sync_copy(xs[d].at[pl.ds(base, C)], xbuf.at[pl.ds(d * C, C)])

        def passA(l):
            p = l & 1
            resf = jnp.float32(RES[l])

            def body_i(i, c2):
                o = i * 16
                ip = []
                for d in range(IN_DIM):
                    pos = xbuf[pl.ds(d * C + o, 16)] * resf
                    ipd = pos.astype(jnp.int32)
                    fracbuf[pl.ds((p * IN_DIM + d) * C + o, 16)] = (
                        pos - ipd.astype(jnp.float32))
                    ip.append(ipd)
                if DENSE[l]:
                    s = RES[l] + 1
                    h0 = (ip[0], ip[0] + 1)
                    h1 = (ip[1] * s, ip[1] * s + s)
                    b2 = ip[2] * (s * s) + (l * T)
                    h2 = (b2, b2 + s * s)
                    t01 = [h0[0] + h1[0], h0[1] + h1[0], h0[0] + h1[1], h0[1] + h1[1]]
                    for c in range(8):
                        e = t01[c & 3] + h2[(c >> 2) & 1]
                        e = e + e
                        idxbuf[pl.ds((p * 16 + 2 * c) * C + o, 16)] = e
                        idxbuf[pl.ds((p * 16 + 2 * c + 1) * C + o, 16)] = e + 1
                else:
                    u = [plsc.bitcast(v, jnp.uint32) for v in ip]
                    h0 = (u[0], u[0] + jnp.uint32(1))
                    m1 = u[1] * jnp.uint32(P1)
                    h1 = (m1, m1 + jnp.uint32(P1))
                    m2 = u[2] * jnp.uint32(P2)
                    h2 = (m2, m2 + jnp.uint32(P2))
                    t01 = [h0[0] ^ h1[0], h0[1] ^ h1[0], h0[0] ^ h1[1], h0[1] ^ h1[1]]
                    mask = jnp.uint32(T - 1)
                    for c in range(8):
                        hv = t01[c & 3] ^ h2[(c >> 2) & 1]
                        e = plsc.bitcast(hv & mask, jnp.int32) + (l * T)
                        e = e + e
                        idxbuf[pl.ds((p * 16 + 2 * c) * C + o, 16)] = e
                        idxbuf[pl.ds((p * 16 + 2 * c + 1) * C + o, 16)] = e + 1
                return c2
            lax.fori_loop(0, NV, body_i, 0)

        def fire(l):
            p = l & 1
            sem = sem0 if p == 0 else sem1
            handles = []
            for k in range(16):
                q = (p * 16 + k) * C
                handles.append(
                    pltpu.async_copy(
                        table.at[idxbuf.at[pl.ds(q, C)]],
                        featbuf.at[pl.ds(q, C)],
                        sem))
            return handles

        def passB(l):
            p = l & 1

            def body_i(i, c2):
                o = i * 16
                f0 = fracbuf[pl.ds((p * IN_DIM + 0) * C + o, 16)]
                f1 = fracbuf[pl.ds((p * IN_DIM + 1) * C + o, 16)]
                f2 = fracbuf[pl.ds((p * IN_DIM + 2) * C + o, 16)]
                g0 = (1.0 - f0, f0)
                g1 = (1.0 - f1, f1)
                g2 = (1.0 - f2, f2)
                w01 = [g0[0] * g1[0], g0[1] * g1[0], g0[0] * g1[1], g0[1] * g1[1]]
                acc0 = jnp.zeros((16,), jnp.float32)
                acc1 = jnp.zeros((16,), jnp.float32)
                for c in range(8):
                    w = w01[c & 3] * g2[(c >> 2) & 1]
                    v0 = featbuf[pl.ds((p * 16 + 2 * c) * C + o, 16)]
                    v1 = featbuf[pl.ds((p * 16 + 2 * c + 1) * C + o, 16)]
                    acc0 = acc0 + w * v0
                    acc1 = acc1 + w * v1
                outbuf[pl.ds((2 * l) * C + o, 16)] = acc0
                outbuf[pl.ds((2 * l + 1) * C + o, 16)] = acc1
                return c2
            lax.fori_loop(0, NV, body_i, 0)

        passA(0)
        pending = fire(0)
        for l in range(1, N_LEVELS + 1):
            if l < N_LEVELS:
                passA(l)
                nxt = fire(l)
            for h in pending:
                h.wait()
            passB(l - 1)
            if l < N_LEVELS:
                pending = nxt

        for k in range(ENC_DIM):
            pltpu.sync_copy(outbuf.at[pl.ds(k * C, C)],
                            out.at[pl.ds(k * N_POINTS + base, C)])
        return carry

    lax.fori_loop(0, NCHUNK, chunk_body, 0)


@jax.jit
def kernel(x, grid):
    x0, x1, x2 = x[:, 0], x[:, 1], x[:, 2]
    table = grid.reshape(N_LEVELS * T * F_PER_LEVEL)
    mesh = plsc.VectorSubcoreMesh(core_axis_name="c", subcore_axis_name="s")
    f = pl.kernel(
        _body,
        out_type=jax.ShapeDtypeStruct((ENC_DIM * N_POINTS,), jnp.float32),
        mesh=mesh,
        compiler_params=pltpu.CompilerParams(needs_layout_passes=False),
        scratch_types=[
            pltpu.VMEM((IN_DIM * C,), jnp.float32),
            pltpu.VMEM((2 * IN_DIM * C,), jnp.float32),
            pltpu.VMEM((2 * 16 * C,), jnp.int32),
            pltpu.VMEM((2 * 16 * C,), jnp.float32),
            pltpu.VMEM((ENC_DIM * C,), jnp.float32),
            pltpu.SemaphoreType.DMA,
            pltpu.SemaphoreType.DMA,
        ],
    )
    enc = f(x0, x1, x2, table).reshape(ENC_DIM, N_POINTS)
    return jnp.concatenate([x, enc.T], axis=-1)


# speedup vs baseline: 4.8863x; 4.8532x over previous
import jax
import jax.numpy as jnp
from jax import lax
from jax.experimental import pallas as pl
from jax.experimental.pallas import tpu as pltpu
from jax.experimental.pallas import tpu_sc as plsc

N_POINTS = 262144
IN_DIM = 3
N_LEVELS = 16
F_PER_LEVEL = 2
LOG2_T = 19
T = 1 << LOG2_T
BASE_RES = 16
PER_LEVEL_SCALE = 1.3819128799
P1 = 2654435761
P2 = 805459861
ENC_DIM = N_LEVELS * F_PER_LEVEL
OUT_DIM = IN_DIM + ENC_DIM


def _res_floor(l):
    import numpy as np
    return int(np.floor(BASE_RES * (PER_LEVEL_SCALE ** l)))


RES = [_res_floor(l) for l in range(N_LEVELS)]
DENSE = [(r + 1) ** 3 <= T for r in RES]

NW = 32
PTS_PER_W = N_POINTS // NW
C = 1024
NCHUNK = PTS_PER_W // C
NV = C // 16


def _body(x0, x1, x2, table, out, xbuf, fracbuf, idxbuf, featbuf, outbuf, sem0, sem1):
    xs = (x0, x1, x2)
    cid = lax.axis_index("c")
    sid = lax.axis_index("s")
    wid = sid * 2 + cid

    def chunk_body(ch, carry):
        base = wid * PTS_PER_W + ch * C
        for d in range(IN_DIM):
            pltpu.sync_copy(xs[d].at[pl.ds(base, C)], xbuf.at[pl.ds(d * C, C)])

        def passA(l):
            p = l & 1
            resf = jnp.float32(RES[l])

            def body_i(i, c2):
                o = i * 16
                ip = []
                for d in range(IN_DIM):
                    pos = xbuf[pl.ds(d * C + o, 16)] * resf
                    ipd = pos.astype(jnp.int32)
                    fracbuf[pl.ds((p * IN_DIM + d) * C + o, 16)] = (
                        pos - ipd.astype(jnp.float32))
                    ip.append(ipd)
                lbase = l * (2 * T)
                if DENSE[l]:
                    s = RES[l] + 1
                    h0 = (ip[0], ip[0] + 1)
                    h1 = (ip[1] * s, ip[1] * s + s)
                    b2 = ip[2] * (s * s)
                    h2 = (b2, b2 + s * s)
                    t01 = [h0[0] + h1[0], h0[1] + h1[0], h0[0] + h1[1], h0[1] + h1[1]]
                    for c in range(8):
                        r = t01[c & 3] + h2[(c >> 2) & 1]
                        lo = r & 127
                        hi = r - lo
                        e = hi + hi + lo + lbase
                        idxbuf[pl.ds((p * 16 + 2 * c) * C + o, 16)] = e
                        idxbuf[pl.ds((p * 16 + 2 * c + 1) * C + o, 16)] = e + 128
                else:
                    u = [plsc.bitcast(v, jnp.uint32) for v in ip]
                    h0 = (u[0], u[0] + jnp.uint32(1))
                    m1 = u[1] * jnp.uint32(P1)
                    h1 = (m1, m1 + jnp.uint32(P1))
                    m2 = u[2] * jnp.uint32(P2)
                    h2 = (m2, m2 + jnp.uint32(P2))
                    t01 = [h0[0] ^ h1[0], h0[1] ^ h1[0], h0[0] ^ h1[1], h0[1] ^ h1[1]]
                    mask_hi = jnp.uint32((T - 1) & ~127)
                    mask_lo = jnp.uint32(127)
                    for c in range(8):
                        hv = t01[c & 3] ^ h2[(c >> 2) & 1]
                        hi = plsc.bitcast(hv & mask_hi, jnp.int32)
                        lo = plsc.bitcast(hv & mask_lo, jnp.int32)
                        e = hi + hi + lo + lbase
                        idxbuf[pl.ds((p * 16 + 2 * c) * C + o, 16)] = e
                        idxbuf[pl.ds((p * 16 + 2 * c + 1) * C + o, 16)] = e + 128
                return c2
            lax.fori_loop(0, NV, body_i, 0)

        def fire(l):
            p = l & 1
            sem = sem0 if p == 0 else sem1
            handles = []
            for k in range(16):
                q = (p * 16 + k) * C
                handles.append(
                    pltpu.async_copy(
                        table.at[idxbuf.at[pl.ds(q, C)]],
                        featbuf.at[pl.ds(q, C)],
                        sem))
            return handles

        def passB(l):
            p = l & 1

            def body_i(i, c2):
                o = i * 16
                f0 = fracbuf[pl.ds((p * IN_DIM + 0) * C + o, 16)]
                f1 = fracbuf[pl.ds((p * IN_DIM + 1) * C + o, 16)]
                f2 = fracbuf[pl.ds((p * IN_DIM + 2) * C + o, 16)]
                g0 = (1.0 - f0, f0)
                g1 = (1.0 - f1, f1)
                g2 = (1.0 - f2, f2)
                w01 = [g0[0] * g1[0], g0[1] * g1[0], g0[0] * g1[1], g0[1] * g1[1]]
                acc0 = jnp.zeros((16,), jnp.float32)
                acc1 = jnp.zeros((16,), jnp.float32)
                for c in range(8):
                    w = w01[c & 3] * g2[(c >> 2) & 1]
                    v0 = featbuf[pl.ds((p * 16 + 2 * c) * C + o, 16)]
                    v1 = featbuf[pl.ds((p * 16 + 2 * c + 1) * C + o, 16)]
                    acc0 = acc0 + w * v0
                    acc1 = acc1 + w * v1
                outbuf[pl.ds((2 * l) * C + o, 16)] = acc0
                outbuf[pl.ds((2 * l + 1) * C + o, 16)] = acc1
                return c2
            lax.fori_loop(0, NV, body_i, 0)

        passA(0)
        pending = fire(0)
        for l in range(1, N_LEVELS + 1):
            if l < N_LEVELS:
                passA(l)
                nxt = fire(l)
            for h in pending:
                h.wait()
            passB(l - 1)
            if l < N_LEVELS:
                pending = nxt

        for k in range(ENC_DIM):
            pltpu.sync_copy(outbuf.at[pl.ds(k * C, C)],
                            out.at[pl.ds(k * N_POINTS + base, C)])
        return carry

    lax.fori_loop(0, NCHUNK, chunk_body, 0)


@jax.jit
def kernel(x, grid):
    x0, x1, x2 = x[:, 0], x[:, 1], x[:, 2]
    table = (grid.reshape(N_LEVELS, T // 128, 128, F_PER_LEVEL)
             .transpose(0, 1, 3, 2)
             .reshape(N_LEVELS * T * F_PER_LEVEL))
    mesh = plsc.VectorSubcoreMesh(core_axis_name="c", subcore_axis_name="s")
    f = pl.kernel(
        _body,
        out_type=jax.ShapeDtypeStruct((ENC_DIM * N_POINTS,), jnp.float32),
        mesh=mesh,
        compiler_params=pltpu.CompilerParams(needs_layout_passes=False),
        scratch_types=[
            pltpu.VMEM((IN_DIM * C,), jnp.float32),
            pltpu.VMEM((2 * IN_DIM * C,), jnp.float32),
            pltpu.VMEM((2 * 16 * C,), jnp.int32),
            pltpu.VMEM((2 * 16 * C,), jnp.float32),
            pltpu.VMEM((ENC_DIM * C,), jnp.float32),
            pltpu.SemaphoreType.DMA,
            pltpu.SemaphoreType.DMA,
        ],
    )
    enc = f(x0, x1, x2, table).reshape(ENC_DIM, N_POINTS)
    return jnp.concatenate([x, enc.T], axis=-1)


# speedup vs baseline: 6.0984x; 1.2481x over previous
import jax
import jax.numpy as jnp
from jax import lax
from jax.experimental import pallas as pl
from jax.experimental.pallas import tpu as pltpu
from jax.experimental.pallas import tpu_sc as plsc

N_POINTS = 262144
IN_DIM = 3
N_LEVELS = 16
F_PER_LEVEL = 2
LOG2_T = 19
T = 1 << LOG2_T
BASE_RES = 16
PER_LEVEL_SCALE = 1.3819128799
P1 = 2654435761
P2 = 805459861
ENC_DIM = N_LEVELS * F_PER_LEVEL
OUT_DIM = IN_DIM + ENC_DIM


def _res_floor(l):
    import numpy as np
    return int(np.floor(BASE_RES * (PER_LEVEL_SCALE ** l)))


RES = [_res_floor(l) for l in range(N_LEVELS)]
DENSE = [(r + 1) ** 3 <= T for r in RES]

NW = 32
PTS_PER_W = N_POINTS // NW
C = 512
NCHUNK = PTS_PER_W // C
NV = C // 16

N_LOCAL = 2
LOCW = [((RES[l] + 1) ** 3 + 127) // 128 * 256 for l in range(N_LOCAL)]
LOC_OFF = [sum(LOCW[:l]) for l in range(N_LOCAL)]
LOC_TOT = sum(LOCW)


def _body(x0, x1, x2, table, out, xbuf, fracbuf, idxbuf, featbuf, outbuf,
          loctab, sem0, sem1):
    xs = (x0, x1, x2)
    cid = lax.axis_index("c")
    sid = lax.axis_index("s")
    wid = sid * 2 + cid

    for l in range(N_LOCAL):
        pltpu.sync_copy(table.at[pl.ds(l * (2 * T), LOCW[l])],
                        loctab.at[pl.ds(LOC_OFF[l], LOCW[l])])

    def chunk_body(ch, carry):
        base = wid * PTS_PER_W + ch * C
        for d in range(IN_DIM):
            pltpu.sync_copy(xs[d].at[pl.ds(base, C)], xbuf.at[pl.ds(d * C, C)])

        def passA(l):
            p = l & 1
            resf = jnp.float32(RES[l])

            def body_i(i, c2):
                o = i * 16
                ip = []
                for d in range(IN_DIM):
                    pos = xbuf[pl.ds(d * C + o, 16)] * resf
                    ipd = pos.astype(jnp.int32)
                    fracbuf[pl.ds((p * IN_DIM + d) * C + o, 16)] = (
                        pos - ipd.astype(jnp.float32))
                    ip.append(ipd)
                lbase = l * (2 * T)
                if DENSE[l]:
                    s = RES[l] + 1
                    h0 = (ip[0], ip[0] + 1)
                    h1 = (ip[1] * s, ip[1] * s + s)
                    b2 = ip[2] * (s * s)
                    h2 = (b2, b2 + s * s)
                    t01 = [h0[0] + h1[0], h0[1] + h1[0], h0[0] + h1[1], h0[1] + h1[1]]
                    for c in range(8):
                        r = t01[c & 3] + h2[(c >> 2) & 1]
                        lo = r & 127
                        hi = r - lo
                        e = hi + hi + lo + lbase
                        idxbuf[pl.ds((p * 16 + 2 * c) * C + o, 16)] = e
                        idxbuf[pl.ds((p * 16 + 2 * c + 1) * C + o, 16)] = e + 128
                else:
                    u = [plsc.bitcast(v, jnp.uint32) for v in ip]
                    h0 = (u[0], u[0] + jnp.uint32(1))
                    m1 = u[1] * jnp.uint32(P1)
                    h1 = (m1, m1 + jnp.uint32(P1))
                    m2 = u[2] * jnp.uint32(P2)
                    h2 = (m2, m2 + jnp.uint32(P2))
                    t01 = [h0[0] ^ h1[0], h0[1] ^ h1[0], h0[0] ^ h1[1], h0[1] ^ h1[1]]
                    mask_hi = jnp.uint32((T - 1) & ~127)
                    mask_lo = jnp.uint32(127)
                    for c in range(8):
                        hv = t01[c & 3] ^ h2[(c >> 2) & 1]
                        hi = plsc.bitcast(hv & mask_hi, jnp.int32)
                        lo = plsc.bitcast(hv & mask_lo, jnp.int32)
                        e = hi + hi + lo + lbase
                        idxbuf[pl.ds((p * 16 + 2 * c) * C + o, 16)] = e
                        idxbuf[pl.ds((p * 16 + 2 * c + 1) * C + o, 16)] = e + 128
                return c2
            lax.fori_loop(0, NV, body_i, 0)

        def fire(l):
            p = l & 1
            sem = sem0 if p == 0 else sem1
            handles = []
            for k in range(16):
                q = (p * 16 + k) * C
                handles.append(
                    pltpu.async_copy(
                        table.at[idxbuf.at[pl.ds(q, C)]],
                        featbuf.at[pl.ds(q, C)],
                        sem))
            return handles

        def passB(l):
            p = l & 1

            def body_i(i, c2):
                o = i * 16
                f0 = fracbuf[pl.ds((p * IN_DIM + 0) * C + o, 16)]
                f1 = fracbuf[pl.ds((p * IN_DIM + 1) * C + o, 16)]
                f2 = fracbuf[pl.ds((p * IN_DIM + 2) * C + o, 16)]
                g0 = (1.0 - f0, f0)
                g1 = (1.0 - f1, f1)
                g2 = (1.0 - f2, f2)
                w01 = [g0[0] * g1[0], g0[1] * g1[0], g0[0] * g1[1], g0[1] * g1[1]]
                acc0 = jnp.zeros((16,), jnp.float32)
                acc1 = jnp.zeros((16,), jnp.float32)
                for c in range(8):
                    w = w01[c & 3] * g2[(c >> 2) & 1]
                    v0 = featbuf[pl.ds((p * 16 + 2 * c) * C + o, 16)]
                    v1 = featbuf[pl.ds((p * 16 + 2 * c + 1) * C + o, 16)]
                    acc0 = acc0 + w * v0
                    acc1 = acc1 + w * v1
                outbuf[pl.ds((2 * l) * C + o, 16)] = acc0
                outbuf[pl.ds((2 * l + 1) * C + o, 16)] = acc1
                return c2
            lax.fori_loop(0, NV, body_i, 0)

        def local_pass(l):
            resf = jnp.float32(RES[l])
            s = RES[l] + 1
            lbase = LOC_OFF[l]

            def body_i(i, c2):
                o = i * 16
                ip = []
                fr = []
                for d in range(IN_DIM):
                    pos = xbuf[pl.ds(d * C + o, 16)] * resf
                    ipd = pos.astype(jnp.int32)
                    fr.append(pos - ipd.astype(jnp.float32))
                    ip.append(ipd)
                h0 = (ip[0], ip[0] + 1)
                h1 = (ip[1] * s, ip[1] * s + s)
                b2 = ip[2] * (s * s)
                h2 = (b2, b2 + s * s)
                t01 = [h0[0] + h1[0], h0[1] + h1[0], h0[0] + h1[1], h0[1] + h1[1]]
                g0 = (1.0 - fr[0], fr[0])
                g1 = (1.0 - fr[1], fr[1])
                g2 = (1.0 - fr[2], fr[2])
                w01 = [g0[0] * g1[0], g0[1] * g1[0], g0[0] * g1[1], g0[1] * g1[1]]
                acc0 = jnp.zeros((16,), jnp.float32)
                acc1 = jnp.zeros((16,), jnp.float32)
                for c in range(8):
                    r = t01[c & 3] + h2[(c >> 2) & 1]
                    lo = r & 127
                    hi = r - lo
                    e = hi + hi + lo + lbase
                    w = w01[c & 3] * g2[(c >> 2) & 1]
                    v0 = plsc.load_gather(loctab, [e])
                    v1 = plsc.load_gather(loctab, [e + 128])
                    acc0 = acc0 + w * v0
                    acc1 = acc1 + w * v1
                outbuf[pl.ds((2 * l) * C + o, 16)] = acc0
                outbuf[pl.ds((2 * l + 1) * C + o, 16)] = acc1
                return c2
            lax.fori_loop(0, NV, body_i, 0)

        passA(N_LOCAL)
        pending = fire(N_LOCAL)
        for l in range(N_LOCAL):
            local_pass(l)
        for l in range(N_LOCAL + 1, N_LEVELS + 1):
            if l < N_LEVELS:
                passA(l)
                nxt = fire(l)
            for h in pending:
                h.wait()
            passB(l - 1)
            if l < N_LEVELS:
                pending = nxt

        for k in range(ENC_DIM):
            pltpu.sync_copy(outbuf.at[pl.ds(k * C, C)],
                            out.at[pl.ds(k * N_POINTS + base, C)])
        return carry

    lax.fori_loop(0, NCHUNK, chunk_body, 0)


@jax.jit
def kernel(x, grid):
    x0, x1, x2 = x[:, 0], x[:, 1], x[:, 2]
    table = (grid.reshape(N_LEVELS, T // 128, 128, F_PER_LEVEL)
             .transpose(0, 1, 3, 2)
             .reshape(N_LEVELS * T * F_PER_LEVEL))
    mesh = plsc.VectorSubcoreMesh(core_axis_name="c", subcore_axis_name="s")
    f = pl.kernel(
        _body,
        out_type=jax.ShapeDtypeStruct((ENC_DIM * N_POINTS,), jnp.float32),
        mesh=mesh,
        compiler_params=pltpu.CompilerParams(needs_layout_passes=False),
        scratch_types=[
            pltpu.VMEM((IN_DIM * C,), jnp.float32),
            pltpu.VMEM((2 * IN_DIM * C,), jnp.float32),
            pltpu.VMEM((2 * 16 * C,), jnp.int32),
            pltpu.VMEM((2 * 16 * C,), jnp.float32),
            pltpu.VMEM((ENC_DIM * C,), jnp.float32),
            pltpu.VMEM((LOC_TOT,), jnp.float32),
            pltpu.SemaphoreType.DMA,
            pltpu.SemaphoreType.DMA,
        ],
    )
    enc = f(x0, x1, x2, table).reshape(ENC_DIM, N_POINTS)
    return jnp.concatenate([x, enc.T], axis=-1)


# speedup vs baseline: 6.5557x; 1.0750x over previous
import jax
import jax.numpy as jnp
from jax import lax
from jax.experimental import pallas as pl
from jax.experimental.pallas import tpu as pltpu
from jax.experimental.pallas import tpu_sc as plsc

N_POINTS = 262144
IN_DIM = 3
N_LEVELS = 16
F_PER_LEVEL = 2
LOG2_T = 19
T = 1 << LOG2_T
BASE_RES = 16
PER_LEVEL_SCALE = 1.3819128799
P1 = 2654435761
P2 = 805459861
ENC_DIM = N_LEVELS * F_PER_LEVEL
OUT_DIM = IN_DIM + ENC_DIM


def _res_floor(l):
    import numpy as np
    return int(np.floor(BASE_RES * (PER_LEVEL_SCALE ** l)))


RES = [_res_floor(l) for l in range(N_LEVELS)]
DENSE = [(r + 1) ** 3 <= T for r in RES]

NW = 32
PTS_PER_W = N_POINTS // NW
C = 256
NCHUNK = PTS_PER_W // C
NV = C // 16

N_LOCAL = 3
LOCW = [((RES[l] + 1) ** 3 + 127) // 128 * 256 for l in range(N_LOCAL)]
LOC_OFF = [sum(LOCW[:l]) for l in range(N_LOCAL)]
LOC_TOT = sum(LOCW)


def _body(x0, x1, x2, table, out, xbuf, fracbuf, idxbuf, featbuf, outbuf,
          loctab, sem0, sem1):
    xs = (x0, x1, x2)
    cid = lax.axis_index("c")
    sid = lax.axis_index("s")
    wid = sid * 2 + cid

    for l in range(N_LOCAL):
        pltpu.sync_copy(table.at[pl.ds(l * (2 * T), LOCW[l])],
                        loctab.at[pl.ds(LOC_OFF[l], LOCW[l])])

    def chunk_body(ch, carry):
        base = wid * PTS_PER_W + ch * C
        for d in range(IN_DIM):
            pltpu.sync_copy(xs[d].at[pl.ds(base, C)], xbuf.at[pl.ds(d * C, C)])

        def passA(l):
            p = l & 1
            resf = jnp.float32(RES[l])

            def body_i(i, c2):
                o = i * 16
                ip = []
                for d in range(IN_DIM):
                    pos = xbuf[pl.ds(d * C + o, 16)] * resf
                    ipd = pos.astype(jnp.int32)
                    fracbuf[pl.ds((p * IN_DIM + d) * C + o, 16)] = (
                        pos - ipd.astype(jnp.float32))
                    ip.append(ipd)
                lbase = l * (2 * T)
                if DENSE[l]:
                    s = RES[l] + 1
                    h0 = (ip[0], ip[0] + 1)
                    h1 = (ip[1] * s, ip[1] * s + s)
                    b2 = ip[2] * (s * s)
                    h2 = (b2, b2 + s * s)
                    t01 = [h0[0] + h1[0], h0[1] + h1[0], h0[0] + h1[1], h0[1] + h1[1]]
                    for c in range(8):
                        r = t01[c & 3] + h2[(c >> 2) & 1]
                        lo = r & 127
                        hi = r - lo
                        e = hi + hi + lo + lbase
                        idxbuf[pl.ds((p * 16 + 2 * c) * C + o, 16)] = e
                        idxbuf[pl.ds((p * 16 + 2 * c + 1) * C + o, 16)] = e + 128
                else:
                    u = [plsc.bitcast(v, jnp.uint32) for v in ip]
                    h0 = (u[0], u[0] + jnp.uint32(1))
                    m1 = u[1] * jnp.uint32(P1)
                    h1 = (m1, m1 + jnp.uint32(P1))
                    m2 = u[2] * jnp.uint32(P2)
                    h2 = (m2, m2 + jnp.uint32(P2))
                    t01 = [h0[0] ^ h1[0], h0[1] ^ h1[0], h0[0] ^ h1[1], h0[1] ^ h1[1]]
                    mask_hi = jnp.uint32((T - 1) & ~127)
                    mask_lo = jnp.uint32(127)
                    for c in range(8):
                        hv = t01[c & 3] ^ h2[(c >> 2) & 1]
                        hi = plsc.bitcast(hv & mask_hi, jnp.int32)
                        lo = plsc.bitcast(hv & mask_lo, jnp.int32)
                        e = hi + hi + lo + lbase
                        idxbuf[pl.ds((p * 16 + 2 * c) * C + o, 16)] = e
                        idxbuf[pl.ds((p * 16 + 2 * c + 1) * C + o, 16)] = e + 128
                return c2
            lax.fori_loop(0, NV, body_i, 0)

        def fire(l):
            p = l & 1
            sem = sem0 if p == 0 else sem1
            handles = []
            for k in range(16):
                q = (p * 16 + k) * C
                handles.append(
                    pltpu.async_copy(
                        table.at[idxbuf.at[pl.ds(q, C)]],
                        featbuf.at[pl.ds(q, C)],
                        sem))
            return handles

        def passB(l):
            p = l & 1

            def body_i(i, c2):
                o = i * 16
                f0 = fracbuf[pl.ds((p * IN_DIM + 0) * C + o, 16)]
                f1 = fracbuf[pl.ds((p * IN_DIM + 1) * C + o, 16)]
                f2 = fracbuf[pl.ds((p * IN_DIM + 2) * C + o, 16)]
                g0 = (1.0 - f0, f0)
                g1 = (1.0 - f1, f1)
                g2 = (1.0 - f2, f2)
                w01 = [g0[0] * g1[0], g0[1] * g1[0], g0[0] * g1[1], g0[1] * g1[1]]
                acc0 = jnp.zeros((16,), jnp.float32)
                acc1 = jnp.zeros((16,), jnp.float32)
                for c in range(8):
                    w = w01[c & 3] * g2[(c >> 2) & 1]
                    v0 = featbuf[pl.ds((p * 16 + 2 * c) * C + o, 16)]
                    v1 = featbuf[pl.ds((p * 16 + 2 * c + 1) * C + o, 16)]
                    acc0 = acc0 + w * v0
                    acc1 = acc1 + w * v1
                outbuf[pl.ds((2 * l) * C + o, 16)] = acc0
                outbuf[pl.ds((2 * l + 1) * C + o, 16)] = acc1
                return c2
            lax.fori_loop(0, NV, body_i, 0)

        def local_pass(l):
            resf = jnp.float32(RES[l])
            s = RES[l] + 1
            lbase = LOC_OFF[l]

            def body_i(i, c2):
                o = i * 16
                ip = []
                fr = []
                for d in range(IN_DIM):
                    pos = xbuf[pl.ds(d * C + o, 16)] * resf
                    ipd = pos.astype(jnp.int32)
                    fr.append(pos - ipd.astype(jnp.float32))
                    ip.append(ipd)
                h0 = (ip[0], ip[0] + 1)
                h1 = (ip[1] * s, ip[1] * s + s)
                b2 = ip[2] * (s * s)
                h2 = (b2, b2 + s * s)
                t01 = [h0[0] + h1[0], h0[1] + h1[0], h0[0] + h1[1], h0[1] + h1[1]]
                g0 = (1.0 - fr[0], fr[0])
                g1 = (1.0 - fr[1], fr[1])
                g2 = (1.0 - fr[2], fr[2])
                w01 = [g0[0] * g1[0], g0[1] * g1[0], g0[0] * g1[1], g0[1] * g1[1]]
                acc0 = jnp.zeros((16,), jnp.float32)
                acc1 = jnp.zeros((16,), jnp.float32)
                for c in range(8):
                    r = t01[c & 3] + h2[(c >> 2) & 1]
                    lo = r & 127
                    hi = r - lo
                    e = hi + hi + lo + lbase
                    w = w01[c & 3] * g2[(c >> 2) & 1]
                    v0 = plsc.load_gather(loctab, [e])
                    v1 = plsc.load_gather(loctab, [e + 128])
                    acc0 = acc0 + w * v0
                    acc1 = acc1 + w * v1
                outbuf[pl.ds((2 * l) * C + o, 16)] = acc0
                outbuf[pl.ds((2 * l + 1) * C + o, 16)] = acc1
                return c2
            lax.fori_loop(0, NV, body_i, 0)

        passA(N_LOCAL)
        pending = fire(N_LOCAL)
        for l in range(N_LOCAL):
            local_pass(l)
        for l in range(N_LOCAL + 1, N_LEVELS + 1):
            if l < N_LEVELS:
                passA(l)
                nxt = fire(l)
            for h in pending:
                h.wait()
            passB(l - 1)
            if l < N_LEVELS:
                pending = nxt

        for k in range(ENC_DIM):
            pltpu.sync_copy(outbuf.at[pl.ds(k * C, C)],
                            out.at[pl.ds(k * N_POINTS + base, C)])
        return carry

    lax.fori_loop(0, NCHUNK, chunk_body, 0)


@jax.jit
def kernel(x, grid):
    x0, x1, x2 = x[:, 0], x[:, 1], x[:, 2]
    table = (grid.reshape(N_LEVELS, T // 128, 128, F_PER_LEVEL)
             .transpose(0, 1, 3, 2)
             .reshape(N_LEVELS * T * F_PER_LEVEL))
    mesh = plsc.VectorSubcoreMesh(core_axis_name="c", subcore_axis_name="s")
    f = pl.kernel(
        _body,
        out_type=jax.ShapeDtypeStruct((ENC_DIM * N_POINTS,), jnp.float32),
        mesh=mesh,
        compiler_params=pltpu.CompilerParams(needs_layout_passes=False),
        scratch_types=[
            pltpu.VMEM((IN_DIM * C,), jnp.float32),
            pltpu.VMEM((2 * IN_DIM * C,), jnp.float32),
            pltpu.VMEM((2 * 16 * C,), jnp.int32),
            pltpu.VMEM((2 * 16 * C,), jnp.float32),
            pltpu.VMEM((ENC_DIM * C,), jnp.float32),
            pltpu.VMEM((LOC_TOT,), jnp.float32),
            pltpu.SemaphoreType.DMA,
            pltpu.SemaphoreType.DMA,
        ],
    )
    enc = f(x0, x1, x2, table).reshape(ENC_DIM, N_POINTS)
    return jnp.concatenate([x, enc.T], axis=-1)


# speedup vs baseline: 6.5851x; 1.0045x over previous
import jax
import jax.numpy as jnp
from jax import lax
from jax.experimental import pallas as pl
from jax.experimental.pallas import tpu as pltpu
from jax.experimental.pallas import tpu_sc as plsc

N_POINTS = 262144
IN_DIM = 3
N_LEVELS = 16
F_PER_LEVEL = 2
LOG2_T = 19
T = 1 << LOG2_T
BASE_RES = 16
PER_LEVEL_SCALE = 1.3819128799
P1 = 2654435761
P2 = 805459861
ENC_DIM = N_LEVELS * F_PER_LEVEL
OUT_DIM = IN_DIM + ENC_DIM


def _res_floor(l):
    import numpy as np
    return int(np.floor(BASE_RES * (PER_LEVEL_SCALE ** l)))


RES = [_res_floor(l) for l in range(N_LEVELS)]
DENSE = [(r + 1) ** 3 <= T for r in RES]

NW = 32
PTS_PER_W = N_POINTS // NW
C = 256
NCHUNK = PTS_PER_W // C
NV = C // 16

N_LOCAL = 3
LOCW = [((RES[l] + 1) ** 3 + 127) // 128 * 256 for l in range(N_LOCAL)]
LOC_OFF = [sum(LOCW[:l]) for l in range(N_LOCAL)]
LOC_TOT = sum(LOCW)


def _body(x0, x1, x2, table, out, xbuf, fracbuf, idxbuf, featbuf, outbuf,
          loctab, sem0, sem1, sem_out):
    xs = (x0, x1, x2)
    cid = lax.axis_index("c")
    sid = lax.axis_index("s")
    wid = sid * 2 + cid

    for l in range(N_LOCAL):
        pltpu.sync_copy(table.at[pl.ds(l * (2 * T), LOCW[l])],
                        loctab.at[pl.ds(LOC_OFF[l], LOCW[l])])

    def chunk_body(ch, carry):
        base = wid * PTS_PER_W + ch * C
        for d in range(IN_DIM):
            pltpu.sync_copy(xs[d].at[pl.ds(base, C)], xbuf.at[pl.ds(d * C, C)])

        def passA(l):
            p = l & 1
            resf = jnp.float32(RES[l])

            def body_i(i, c2):
                o = i * 16
                ip = []
                for d in range(IN_DIM):
                    pos = xbuf[pl.ds(d * C + o, 16)] * resf
                    ipd = pos.astype(jnp.int32)
                    fracbuf[pl.ds((p * IN_DIM + d) * C + o, 16)] = (
                        pos - ipd.astype(jnp.float32))
                    ip.append(ipd)
                lbase = l * (2 * T)
                if DENSE[l]:
                    s = RES[l] + 1
                    h0 = (ip[0], ip[0] + 1)
                    h1 = (ip[1] * s, ip[1] * s + s)
                    b2 = ip[2] * (s * s)
                    h2 = (b2, b2 + s * s)
                    t01 = [h0[0] + h1[0], h0[1] + h1[0], h0[0] + h1[1], h0[1] + h1[1]]
                    for c in range(8):
                        r = t01[c & 3] + h2[(c >> 2) & 1]
                        lo = r & 127
                        hi = r - lo
                        e = hi + hi + lo + lbase
                        idxbuf[pl.ds((p * 16 + 2 * c) * C + o, 16)] = e
                        idxbuf[pl.ds((p * 16 + 2 * c + 1) * C + o, 16)] = e + 128
                else:
                    u = [plsc.bitcast(v, jnp.uint32) for v in ip]
                    h0 = (u[0], u[0] + jnp.uint32(1))
                    m1 = u[1] * jnp.uint32(P1)
                    h1 = (m1, m1 + jnp.uint32(P1))
                    m2 = u[2] * jnp.uint32(P2)
                    h2 = (m2, m2 + jnp.uint32(P2))
                    t01 = [h0[0] ^ h1[0], h0[1] ^ h1[0], h0[0] ^ h1[1], h0[1] ^ h1[1]]
                    mask_hi = jnp.uint32((T - 1) & ~127)
                    mask_lo = jnp.uint32(127)
                    for c in range(8):
                        hv = t01[c & 3] ^ h2[(c >> 2) & 1]
                        hi = plsc.bitcast(hv & mask_hi, jnp.int32)
                        lo = plsc.bitcast(hv & mask_lo, jnp.int32)
                        e = hi + hi + lo + lbase
                        idxbuf[pl.ds((p * 16 + 2 * c) * C + o, 16)] = e
                        idxbuf[pl.ds((p * 16 + 2 * c + 1) * C + o, 16)] = e + 128
                return c2
            lax.fori_loop(0, NV, body_i, 0)

        def fire(l):
            p = l & 1
            sem = sem0 if p == 0 else sem1
            handles = []
            for k in range(16):
                q = (p * 16 + k) * C
                handles.append(
                    pltpu.async_copy(
                        table.at[idxbuf.at[pl.ds(q, C)]],
                        featbuf.at[pl.ds(q, C)],
                        sem))
            return handles

        def passB(l):
            p = l & 1

            def body_i(i, c2):
                o = i * 16
                f0 = fracbuf[pl.ds((p * IN_DIM + 0) * C + o, 16)]
                f1 = fracbuf[pl.ds((p * IN_DIM + 1) * C + o, 16)]
                f2 = fracbuf[pl.ds((p * IN_DIM + 2) * C + o, 16)]
                g0 = (1.0 - f0, f0)
                g1 = (1.0 - f1, f1)
                g2 = (1.0 - f2, f2)
                w01 = [g0[0] * g1[0], g0[1] * g1[0], g0[0] * g1[1], g0[1] * g1[1]]
                acc0 = jnp.zeros((16,), jnp.float32)
                acc1 = jnp.zeros((16,), jnp.float32)
                for c in range(8):
                    w = w01[c & 3] * g2[(c >> 2) & 1]
                    v0 = featbuf[pl.ds((p * 16 + 2 * c) * C + o, 16)]
                    v1 = featbuf[pl.ds((p * 16 + 2 * c + 1) * C + o, 16)]
                    acc0 = acc0 + w * v0
                    acc1 = acc1 + w * v1
                outbuf[pl.ds((2 * l) * C + o, 16)] = acc0
                outbuf[pl.ds((2 * l + 1) * C + o, 16)] = acc1
                return c2
            lax.fori_loop(0, NV, body_i, 0)

        def local_pass(l):
            resf = jnp.float32(RES[l])
            s = RES[l] + 1
            lbase = LOC_OFF[l]

            def body_i(i, c2):
                o = i * 16
                ip = []
                fr = []
                for d in range(IN_DIM):
                    pos = xbuf[pl.ds(d * C + o, 16)] * resf
                    ipd = pos.astype(jnp.int32)
                    fr.append(pos - ipd.astype(jnp.float32))
                    ip.append(ipd)
                h0 = (ip[0], ip[0] + 1)
                h1 = (ip[1] * s, ip[1] * s + s)
                b2 = ip[2] * (s * s)
                h2 = (b2, b2 + s * s)
                t01 = [h0[0] + h1[0], h0[1] + h1[0], h0[0] + h1[1], h0[1] + h1[1]]
                g0 = (1.0 - fr[0], fr[0])
                g1 = (1.0 - fr[1], fr[1])
                g2 = (1.0 - fr[2], fr[2])
                w01 = [g0[0] * g1[0], g0[1] * g1[0], g0[0] * g1[1], g0[1] * g1[1]]
                acc0 = jnp.zeros((16,), jnp.float32)
                acc1 = jnp.zeros((16,), jnp.float32)
                for c in range(8):
                    r = t01[c & 3] + h2[(c >> 2) & 1]
                    lo = r & 127
                    hi = r - lo
                    e = hi + hi + lo + lbase
                    w = w01[c & 3] * g2[(c >> 2) & 1]
                    v0 = plsc.load_gather(loctab, [e])
                    v1 = plsc.load_gather(loctab, [e + 128])
                    acc0 = acc0 + w * v0
                    acc1 = acc1 + w * v1
                outbuf[pl.ds((2 * l) * C + o, 16)] = acc0
                outbuf[pl.ds((2 * l + 1) * C + o, 16)] = acc1
                return c2
            lax.fori_loop(0, NV, body_i, 0)

        passA(N_LOCAL)
        pending = fire(N_LOCAL)

        @pl.when(ch != 0)
        def _():
            for k in range(ENC_DIM):
                pltpu.make_async_copy(
                    outbuf.at[pl.ds(k * C, C)],
                    out.at[pl.ds(k * N_POINTS, C)], sem_out).wait()

        for l in range(N_LOCAL):
            local_pass(l)
        for l in range(N_LOCAL + 1, N_LEVELS + 1):
            if l < N_LEVELS:
                passA(l)
                nxt = fire(l)
            for h in pending:
                h.wait()
            passB(l - 1)
            if l < N_LEVELS:
                pending = nxt

        for k in range(ENC_DIM):
            pltpu.async_copy(outbuf.at[pl.ds(k * C, C)],
                             out.at[pl.ds(k * N_POINTS + base, C)], sem_out)
        return carry

    lax.fori_loop(0, NCHUNK, chunk_body, 0)
    for k in range(ENC_DIM):
        pltpu.make_async_copy(
            outbuf.at[pl.ds(k * C, C)],
            out.at[pl.ds(k * N_POINTS, C)], sem_out).wait()


@jax.jit
def kernel(x, grid):
    x0, x1, x2 = x[:, 0], x[:, 1], x[:, 2]
    table = (grid.reshape(N_LEVELS, T // 128, 128, F_PER_LEVEL)
             .transpose(0, 1, 3, 2)
             .reshape(N_LEVELS * T * F_PER_LEVEL))
    mesh = plsc.VectorSubcoreMesh(core_axis_name="c", subcore_axis_name="s")
    f = pl.kernel(
        _body,
        out_type=jax.ShapeDtypeStruct((ENC_DIM * N_POINTS,), jnp.float32),
        mesh=mesh,
        compiler_params=pltpu.CompilerParams(needs_layout_passes=False),
        scratch_types=[
            pltpu.VMEM((IN_DIM * C,), jnp.float32),
            pltpu.VMEM((2 * IN_DIM * C,), jnp.float32),
            pltpu.VMEM((2 * 16 * C,), jnp.int32),
            pltpu.VMEM((2 * 16 * C,), jnp.float32),
            pltpu.VMEM((ENC_DIM * C,), jnp.float32),
            pltpu.VMEM((LOC_TOT,), jnp.float32),
            pltpu.SemaphoreType.DMA,
            pltpu.SemaphoreType.DMA,
            pltpu.SemaphoreType.DMA,
        ],
    )
    enc = f(x0, x1, x2, table).reshape(ENC_DIM, N_POINTS)
    return jnp.concatenate([x, enc.T], axis=-1)


# speedup vs baseline: 11.1551x; 1.6940x over previous
import jax
import jax.numpy as jnp
from jax import lax
from jax.experimental import pallas as pl
from jax.experimental.pallas import tpu as pltpu
from jax.experimental.pallas import tpu_sc as plsc

N_POINTS = 262144
IN_DIM = 3
N_LEVELS = 16
F_PER_LEVEL = 2
LOG2_T = 19
T = 1 << LOG2_T
BASE_RES = 16
PER_LEVEL_SCALE = 1.3819128799
P1 = 2654435761
P2 = 805459861
ENC_DIM = N_LEVELS * F_PER_LEVEL
OUT_DIM = IN_DIM + ENC_DIM


def _res_floor(l):
    import numpy as np
    return int(np.floor(BASE_RES * (PER_LEVEL_SCALE ** l)))


RES = [_res_floor(l) for l in range(N_LEVELS)]
DENSE = [(r + 1) ** 3 <= T for r in RES]

NW = 32
PTS_PER_W = N_POINTS // NW
C = 256
NCHUNK = PTS_PER_W // C
NV = C // 16

N_LOCAL = 3
LOCW = [((RES[l] + 1) ** 3 + 7) // 8 * 8 for l in range(N_LOCAL)]
LOC_OFF = [sum(LOCW[:l]) for l in range(N_LOCAL)]
LOC_TOT = sum(LOCW)

FMASK = -65536


def _body(x0, x1, x2, *refs):
    tabs = refs[:N_LEVELS]
    out = refs[N_LEVELS]
    (xbuf, fracbuf, idxbuf, featbuf, outbuf, loctab,
     sem0, sem1, sem_out) = refs[N_LEVELS + 1:]
    xs = (x0, x1, x2)
    cid = lax.axis_index("c")
    sid = lax.axis_index("s")
    wid = sid * 2 + cid

    for l in range(N_LOCAL):
        pltpu.sync_copy(tabs[l].at[pl.ds(0, LOCW[l])],
                        loctab.at[pl.ds(LOC_OFF[l], LOCW[l])])

    def chunk_body(ch, carry):
        base = wid * PTS_PER_W + ch * C
        for d in range(IN_DIM):
            pltpu.sync_copy(xs[d].at[pl.ds(base, C)], xbuf.at[pl.ds(d * C, C)])

        def passA(l):
            p = l & 1
            resf = jnp.float32(RES[l])

            def body_i(i, c2):
                o = i * 16
                ip = []
                for d in range(IN_DIM):
                    pos = xbuf[pl.ds(d * C + o, 16)] * resf
                    ipd = pos.astype(jnp.int32)
                    fracbuf[pl.ds((p * IN_DIM + d) * C + o, 16)] = (
                        pos - ipd.astype(jnp.float32))
                    ip.append(ipd)
                if DENSE[l]:
                    s = RES[l] + 1
                    h0 = (ip[0], ip[0] + 1)
                    h1 = (ip[1] * s, ip[1] * s + s)
                    b2 = ip[2] * (s * s)
                    h2 = (b2, b2 + s * s)
                    t01 = [h0[0] + h1[0], h0[1] + h1[0], h0[0] + h1[1], h0[1] + h1[1]]
                    for c in range(8):
                        idxbuf[pl.ds((p * 8 + c) * C + o, 16)] = (
                            t01[c & 3] + h2[(c >> 2) & 1])
                else:
                    u = [plsc.bitcast(v, jnp.uint32) for v in ip]
                    h0 = (u[0], u[0] + jnp.uint32(1))
                    m1 = u[1] * jnp.uint32(P1)
                    h1 = (m1, m1 + jnp.uint32(P1))
                    m2 = u[2] * jnp.uint32(P2)
                    h2 = (m2, m2 + jnp.uint32(P2))
                    t01 = [h0[0] ^ h1[0], h0[1] ^ h1[0], h0[0] ^ h1[1], h0[1] ^ h1[1]]
                    mask = jnp.uint32(T - 1)
                    for c in range(8):
                        hv = t01[c & 3] ^ h2[(c >> 2) & 1]
                        idxbuf[pl.ds((p * 8 + c) * C + o, 16)] = (
                            plsc.bitcast(hv & mask, jnp.int32))
                return c2
            lax.fori_loop(0, NV, body_i, 0)

        def fire(l):
            p = l & 1
            sem = sem0 if p == 0 else sem1
            handles = []
            for c in range(8):
                q = (p * 8 + c) * C
                handles.append(
                    pltpu.async_copy(
                        tabs[l].at[idxbuf.at[pl.ds(q, C)]],
                        featbuf.at[pl.ds(q, C)],
                        sem))
            return handles

        def passB(l):
            p = l & 1

            def body_i(i, c2):
                o = i * 16
                f0 = fracbuf[pl.ds((p * IN_DIM + 0) * C + o, 16)]
                f1 = fracbuf[pl.ds((p * IN_DIM + 1) * C + o, 16)]
                f2 = fracbuf[pl.ds((p * IN_DIM + 2) * C + o, 16)]
                g0 = (1.0 - f0, f0)
                g1 = (1.0 - f1, f1)
                g2 = (1.0 - f2, f2)
                w01 = [g0[0] * g1[0], g0[1] * g1[0], g0[0] * g1[1], g0[1] * g1[1]]
                acc0 = jnp.zeros((16,), jnp.float32)
                acc1 = jnp.zeros((16,), jnp.float32)
                for c in range(8):
                    w = w01[c & 3] * g2[(c >> 2) & 1]
                    v = featbuf[pl.ds((p * 8 + c) * C + o, 16)]
                    v0 = plsc.bitcast(v << 16, jnp.float32)
                    v1 = plsc.bitcast(v & FMASK, jnp.float32)
                    acc0 = acc0 + w * v0
                    acc1 = acc1 + w * v1
                outbuf[pl.ds((2 * l) * C + o, 16)] = acc0
                outbuf[pl.ds((2 * l + 1) * C + o, 16)] = acc1
                return c2
            lax.fori_loop(0, NV, body_i, 0)

        def local_pass(l):
            resf = jnp.float32(RES[l])
            s = RES[l] + 1
            lbase = LOC_OFF[l]

            def body_i(i, c2):
                o = i * 16
                ip = []
                fr = []
                for d in range(IN_DIM):
                    pos = xbuf[pl.ds(d * C + o, 16)] * resf
                    ipd = pos.astype(jnp.int32)
                    fr.append(pos - ipd.astype(jnp.float32))
                    ip.append(ipd)
                h0 = (ip[0], ip[0] + 1)
                h1 = (ip[1] * s, ip[1] * s + s)
                b2 = ip[2] * (s * s) + lbase
                h2 = (b2, b2 + s * s)
                t01 = [h0[0] + h1[0], h0[1] + h1[0], h0[0] + h1[1], h0[1] + h1[1]]
                g0 = (1.0 - fr[0], fr[0])
                g1 = (1.0 - fr[1], fr[1])
                g2 = (1.0 - fr[2], fr[2])
                w01 = [g0[0] * g1[0], g0[1] * g1[0], g0[0] * g1[1], g0[1] * g1[1]]
                acc0 = jnp.zeros((16,), jnp.float32)
                acc1 = jnp.zeros((16,), jnp.float32)
                for c in range(8):
                    e = t01[c & 3] + h2[(c >> 2) & 1]
                    w = w01[c & 3] * g2[(c >> 2) & 1]
                    v = plsc.load_gather(loctab, [e])
                    v0 = plsc.bitcast(v << 16, jnp.float32)
                    v1 = plsc.bitcast(v & FMASK, jnp.float32)
                    acc0 = acc0 + w * v0
                    acc1 = acc1 + w * v1
                outbuf[pl.ds((2 * l) * C + o, 16)] = acc0
                outbuf[pl.ds((2 * l + 1) * C + o, 16)] = acc1
                return c2
            lax.fori_loop(0, NV, body_i, 0)

        passA(N_LOCAL)
        pending = fire(N_LOCAL)

        @pl.when(ch != 0)
        def _():
            for k in range(ENC_DIM):
                pltpu.make_async_copy(
                    outbuf.at[pl.ds(k * C, C)],
                    out.at[pl.ds(k * N_POINTS, C)], sem_out).wait()

        for l in range(N_LOCAL):
            local_pass(l)
        for l in range(N_LOCAL + 1, N_LEVELS + 1):
            if l < N_LEVELS:
                passA(l)
                nxt = fire(l)
            for h in pending:
                h.wait()
            passB(l - 1)
            if l < N_LEVELS:
                pending = nxt

        for k in range(ENC_DIM):
            pltpu.async_copy(outbuf.at[pl.ds(k * C, C)],
                             out.at[pl.ds(k * N_POINTS + base, C)], sem_out)
        return carry

    lax.fori_loop(0, NCHUNK, chunk_body, 0)
    for k in range(ENC_DIM):
        pltpu.make_async_copy(
            outbuf.at[pl.ds(k * C, C)],
            out.at[pl.ds(k * N_POINTS, C)], sem_out).wait()


@jax.jit
def kernel(x, grid):
    x0, x1, x2 = x[:, 0], x[:, 1], x[:, 2]
    gb = lax.bitcast_convert_type(grid.astype(jnp.bfloat16), jnp.uint16)
    tabs = [
        (gb[l, :, 0].astype(jnp.uint32)
         | (gb[l, :, 1].astype(jnp.uint32) << 16)).astype(jnp.int32)
        for l in range(N_LEVELS)
    ]
    mesh = plsc.VectorSubcoreMesh(core_axis_name="c", subcore_axis_name="s")
    f = pl.kernel(
        _body,
        out_type=jax.ShapeDtypeStruct((ENC_DIM * N_POINTS,), jnp.float32),
        mesh=mesh,
        compiler_params=pltpu.CompilerParams(needs_layout_passes=False),
        scratch_types=[
            pltpu.VMEM((IN_DIM * C,), jnp.float32),
            pltpu.VMEM((2 * IN_DIM * C,), jnp.float32),
            pltpu.VMEM((2 * 8 * C,), jnp.int32),
            pltpu.VMEM((2 * 8 * C,), jnp.int32),
            pltpu.VMEM((ENC_DIM * C,), jnp.float32),
            pltpu.VMEM((LOC_TOT,), jnp.int32),
            pltpu.SemaphoreType.DMA,
            pltpu.SemaphoreType.DMA,
            pltpu.SemaphoreType.DMA,
        ],
    )
    enc = f(x0, x1, x2, *tabs).reshape(ENC_DIM, N_POINTS)
    return jnp.concatenate([x, enc.T], axis=-1)


# speedup vs baseline: 11.1879x; 1.0029x over previous
import jax
import jax.numpy as jnp
from jax import lax
from jax.experimental import pallas as pl
from jax.experimental.pallas import tpu as pltpu
from jax.experimental.pallas import tpu_sc as plsc

N_POINTS = 262144
IN_DIM = 3
N_LEVELS = 16
F_PER_LEVEL = 2
LOG2_T = 19
T = 1 << LOG2_T
BASE_RES = 16
PER_LEVEL_SCALE = 1.3819128799
P1 = 2654435761
P2 = 805459861
ENC_DIM = N_LEVELS * F_PER_LEVEL
OUT_DIM = IN_DIM + ENC_DIM


def _res_floor(l):
    import numpy as np
    return int(np.floor(BASE_RES * (PER_LEVEL_SCALE ** l)))


RES = [_res_floor(l) for l in range(N_LEVELS)]
DENSE = [(r + 1) ** 3 <= T for r in RES]

NW = 32
PTS_PER_W = N_POINTS // NW
C = 512
NCHUNK = PTS_PER_W // C
NV = C // 16

N_LOCAL = 3
LOCW = [((RES[l] + 1) ** 3 + 7) // 8 * 8 for l in range(N_LOCAL)]
LOC_OFF = [sum(LOCW[:l]) for l in range(N_LOCAL)]
LOC_TOT = sum(LOCW)

FMASK = -65536


def _body(x0, x1, x2, *refs):
    tabs = refs[:N_LEVELS]
    out = refs[N_LEVELS]
    (xbuf, fracbuf, idxbuf, featbuf, outbuf, loctab,
     sem0, sem1, sem_out) = refs[N_LEVELS + 1:]
    xs = (x0, x1, x2)
    cid = lax.axis_index("c")
    sid = lax.axis_index("s")
    wid = sid * 2 + cid

    for l in range(N_LOCAL):
        pltpu.sync_copy(tabs[l].at[pl.ds(0, LOCW[l])],
                        loctab.at[pl.ds(LOC_OFF[l], LOCW[l])])

    def chunk_body(ch, carry):
        base = wid * PTS_PER_W + ch * C
        for d in range(IN_DIM):
            pltpu.sync_copy(xs[d].at[pl.ds(base, C)], xbuf.at[pl.ds(d * C, C)])

        def passA(l):
            p = l & 1
            resf = jnp.float32(RES[l])

            def body_i(i, c2):
                o = i * 16
                ip = []
                for d in range(IN_DIM):
                    pos = xbuf[pl.ds(d * C + o, 16)] * resf
                    ipd = pos.astype(jnp.int32)
                    fracbuf[pl.ds((p * IN_DIM + d) * C + o, 16)] = (
                        pos - ipd.astype(jnp.float32))
                    ip.append(ipd)
                if DENSE[l]:
                    s = RES[l] + 1
                    h0 = (ip[0], ip[0] + 1)
                    h1 = (ip[1] * s, ip[1] * s + s)
                    b2 = ip[2] * (s * s)
                    h2 = (b2, b2 + s * s)
                    t01 = [h0[0] + h1[0], h0[1] + h1[0], h0[0] + h1[1], h0[1] + h1[1]]
                    for c in range(8):
                        idxbuf[pl.ds((p * 8 + c) * C + o, 16)] = (
                            t01[c & 3] + h2[(c >> 2) & 1])
                else:
                    u = [plsc.bitcast(v, jnp.uint32) for v in ip]
                    h0 = (u[0], u[0] + jnp.uint32(1))
                    m1 = u[1] * jnp.uint32(P1)
                    h1 = (m1, m1 + jnp.uint32(P1))
                    m2 = u[2] * jnp.uint32(P2)
                    h2 = (m2, m2 + jnp.uint32(P2))
                    t01 = [h0[0] ^ h1[0], h0[1] ^ h1[0], h0[0] ^ h1[1], h0[1] ^ h1[1]]
                    mask = jnp.uint32(T - 1)
                    for c in range(8):
                        hv = t01[c & 3] ^ h2[(c >> 2) & 1]
                        idxbuf[pl.ds((p * 8 + c) * C + o, 16)] = (
                            plsc.bitcast(hv & mask, jnp.int32))
                return c2
            lax.fori_loop(0, NV, body_i, 0)

        def fire(l):
            p = l & 1
            sem = sem0 if p == 0 else sem1
            handles = []
            for c in range(8):
                q = (p * 8 + c) * C
                handles.append(
                    pltpu.async_copy(
                        tabs[l].at[idxbuf.at[pl.ds(q, C)]],
                        featbuf.at[pl.ds(q, C)],
                        sem))
            return handles

        def passB(l):
            p = l & 1

            def body_i(i, c2):
                o = i * 16
                f0 = fracbuf[pl.ds((p * IN_DIM + 0) * C + o, 16)]
                f1 = fracbuf[pl.ds((p * IN_DIM + 1) * C + o, 16)]
                f2 = fracbuf[pl.ds((p * IN_DIM + 2) * C + o, 16)]
                g0 = (1.0 - f0, f0)
                g1 = (1.0 - f1, f1)
                g2 = (1.0 - f2, f2)
                w01 = [g0[0] * g1[0], g0[1] * g1[0], g0[0] * g1[1], g0[1] * g1[1]]
                acc0 = jnp.zeros((16,), jnp.float32)
                acc1 = jnp.zeros((16,), jnp.float32)
                for c in range(8):
                    w = w01[c & 3] * g2[(c >> 2) & 1]
                    v = featbuf[pl.ds((p * 8 + c) * C + o, 16)]
                    v0 = plsc.bitcast(v << 16, jnp.float32)
                    v1 = plsc.bitcast(v & FMASK, jnp.float32)
                    acc0 = acc0 + w * v0
                    acc1 = acc1 + w * v1
                outbuf[pl.ds((2 * l) * C + o, 16)] = acc0
                outbuf[pl.ds((2 * l + 1) * C + o, 16)] = acc1
                return c2
            lax.fori_loop(0, NV, body_i, 0)

        def local_pass(l):
            resf = jnp.float32(RES[l])
            s = RES[l] + 1
            lbase = LOC_OFF[l]

            def body_i(i, c2):
                o = i * 16
                ip = []
                fr = []
                for d in range(IN_DIM):
                    pos = xbuf[pl.ds(d * C + o, 16)] * resf
                    ipd = pos.astype(jnp.int32)
                    fr.append(pos - ipd.astype(jnp.float32))
                    ip.append(ipd)
                h0 = (ip[0], ip[0] + 1)
                h1 = (ip[1] * s, ip[1] * s + s)
                b2 = ip[2] * (s * s) + lbase
                h2 = (b2, b2 + s * s)
                t01 = [h0[0] + h1[0], h0[1] + h1[0], h0[0] + h1[1], h0[1] + h1[1]]
                g0 = (1.0 - fr[0], fr[0])
                g1 = (1.0 - fr[1], fr[1])
                g2 = (1.0 - fr[2], fr[2])
                w01 = [g0[0] * g1[0], g0[1] * g1[0], g0[0] * g1[1], g0[1] * g1[1]]
                acc0 = jnp.zeros((16,), jnp.float32)
                acc1 = jnp.zeros((16,), jnp.float32)
                for c in range(8):
                    e = t01[c & 3] + h2[(c >> 2) & 1]
                    w = w01[c & 3] * g2[(c >> 2) & 1]
                    v = plsc.load_gather(loctab, [e])
                    v0 = plsc.bitcast(v << 16, jnp.float32)
                    v1 = plsc.bitcast(v & FMASK, jnp.float32)
                    acc0 = acc0 + w * v0
                    acc1 = acc1 + w * v1
                outbuf[pl.ds((2 * l) * C + o, 16)] = acc0
                outbuf[pl.ds((2 * l + 1) * C + o, 16)] = acc1
                return c2
            lax.fori_loop(0, NV, body_i, 0)

        passA(N_LOCAL)
        pending = fire(N_LOCAL)

        @pl.when(ch != 0)
        def _():
            for k in range(ENC_DIM):
                pltpu.make_async_copy(
                    outbuf.at[pl.ds(k * C, C)],
                    out.at[pl.ds(k * N_POINTS, C)], sem_out).wait()

        for l in range(N_LOCAL):
            local_pass(l)
        for l in range(N_LOCAL + 1, N_LEVELS + 1):
            if l < N_LEVELS:
                passA(l)
                nxt = fire(l)
            for h in pending:
                h.wait()
            passB(l - 1)
            if l < N_LEVELS:
                pending = nxt

        for k in range(ENC_DIM):
            pltpu.async_copy(outbuf.at[pl.ds(k * C, C)],
                             out.at[pl.ds(k * N_POINTS + base, C)], sem_out)
        return carry

    lax.fori_loop(0, NCHUNK, chunk_body, 0)
    for k in range(ENC_DIM):
        pltpu.make_async_copy(
            outbuf.at[pl.ds(k * C, C)],
            out.at[pl.ds(k * N_POINTS, C)], sem_out).wait()


@jax.jit
def kernel(x, grid):
    x0, x1, x2 = x[:, 0], x[:, 1], x[:, 2]
    gb = lax.bitcast_convert_type(grid.astype(jnp.bfloat16), jnp.uint16)
    tabs = [
        lax.bitcast_convert_type(
            gb[l, :, 0].astype(jnp.uint32)
            | (gb[l, :, 1].astype(jnp.uint32) << 16), jnp.int32)
        for l in range(N_LEVELS)
    ]
    mesh = plsc.VectorSubcoreMesh(core_axis_name="c", subcore_axis_name="s")
    f = pl.kernel(
        _body,
        out_type=jax.ShapeDtypeStruct((ENC_DIM * N_POINTS,), jnp.float32),
        mesh=mesh,
        compiler_params=pltpu.CompilerParams(needs_layout_passes=False),
        scratch_types=[
            pltpu.VMEM((IN_DIM * C,), jnp.float32),
            pltpu.VMEM((2 * IN_DIM * C,), jnp.float32),
            pltpu.VMEM((2 * 8 * C,), jnp.int32),
            pltpu.VMEM((2 * 8 * C,), jnp.int32),
            pltpu.VMEM((ENC_DIM * C,), jnp.float32),
            pltpu.VMEM((LOC_TOT,), jnp.int32),
            pltpu.SemaphoreType.DMA,
            pltpu.SemaphoreType.DMA,
            pltpu.SemaphoreType.DMA,
        ],
    )
    enc = f(x0, x1, x2, *tabs).reshape(ENC_DIM, N_POINTS)
    return jnp.concatenate([x, enc.T], axis=-1)
